# baseline (speedup 1.0000x reference)
import functools

import jax
import jax.numpy as jnp
import numpy as np
from jax import lax
from jax.experimental import pallas as pl
from jax.experimental.pallas import tpu as pltpu
from jax.experimental.pallas import tpu_sc as plsc

NC = 2
NS = 16
NW = NC * NS
CH = 128
GH = 64
BN_EPS = 1e-3


def _sc_edge_aggregate(x, edges_m, src_pad, dst_pad, n_acc, k):
    n, D = x.shape
    rc = edges_m.shape[1]
    rpt = n_acc // NS
    kp = k // 2
    bw = NW - 1
    assert bw * k < rc <= NW * k
    rib = (rc - bw * k) // 8 * 8
    mesh = plsc.VectorSubcoreMesh(core_axis_name="c", subcore_axis_name="s")

    @functools.partial(
        pl.kernel,
        out_type=(
            jax.ShapeDtypeStruct((n_acc, D), jnp.float32),
            jax.ShapeDtypeStruct((n_acc, D), jnp.float32),
        ),
        mesh=mesh,
        scratch_types=[
            pltpu.VMEM((kp, CH), jnp.int32),
            pltpu.VMEM((kp, CH), jnp.int32),
            pltpu.VMEM((2 * CH, D), jnp.float32),
            pltpu.SemaphoreType.DMA,
            pltpu.SemaphoreType.DMA,
            pltpu.SemaphoreType.DMA,
            pltpu.SemaphoreType.DMA,
            pltpu.SemaphoreType.DMA,
            pltpu.SemaphoreType.DMA,
            pltpu.SemaphoreType.DMA,
            pltpu.VMEM_SHARED((n_acc, D), jnp.float32),
        ],
    )
    def agg(x_hbm, edges_hbm, srcp_hbm, dstp_hbm,
            out0_hbm, out1_hbm,
            src_v, dst_v, rows_v, isem, gsem0, gsem1, gsem2, gsem3,
            ssem0, ssem1, acc_sh):
        cid = lax.axis_index("c")
        sid = lax.axis_index("s")
        wid = sid * NC + cid

        def stage(phase, row, pad_hbm, buf, op):
            r_lo = min(rib, phase * kp)
            r_ph = min(rib - r_lo, kp)
            p_lo = phase * kp - r_lo

            @pl.when(wid < bw)
            def _():
                op(edges_hbm.at[row, pl.ds(wid * k + phase * kp, kp)], buf)

            @pl.when(wid == bw)
            def _():
                if r_ph:
                    op(edges_hbm.at[row, pl.ds(bw * k + r_lo, r_ph)],
                       buf.at[pl.ds(0, r_ph)])
                if kp - r_ph:
                    op(pad_hbm.at[pl.ds(p_lo, kp - r_ph)],
                       buf.at[pl.ds(r_ph, kp - r_ph)])

        def istart(s, d):
            pltpu.async_copy(s, d, isem)

        def iwait(s, d):
            pltpu.make_async_copy(s, d, isem).wait()

        stage(0, 0, srcp_hbm, src_v, istart)
        stage(0, 1, dstp_hbm, dst_v, istart)

        zvec = jnp.zeros((16,), jnp.float32)

        def zrow(i, carry):
            for l in range(D // 16):
                rows_v[i, pl.ds(l * 16, 16)] = zvec
            return carry

        lax.fori_loop(0, CH, zrow, 0)
        for r in range(rpt // CH):
            pltpu.sync_copy(rows_v.at[pl.ds(0, CH)],
                            acc_sh.at[pl.ds(sid * rpt + r * CH, CH)])

        stage(0, 0, srcp_hbm, src_v, iwait)
        stage(0, 1, dstp_hbm, dst_v, iwait)
        plsc.subcore_barrier()

        def gather(j, half, seg, sem):
            pltpu.async_copy(x_hbm.at[src_v.at[j, pl.ds(half * GH, GH)]],
                             rows_v.at[pl.ds(seg * GH, GH)], sem)

        def gather_wait(j, half, seg, sem):
            pltpu.make_async_copy(
                x_hbm.at[src_v.at[j, pl.ds(half * GH, GH)]],
                rows_v.at[pl.ds(seg * GH, GH)], sem).wait()

        def scatter(j, pair, sem):
            pltpu.async_copy(rows_v.at[pl.ds(pair * CH, CH)],
                             acc_sh.at[dst_v.at[j]], sem, add=True)

        def scatter_wait(j, pair, sem):
            pltpu.make_async_copy(rows_v.at[pl.ds(pair * CH, CH)],
                                  acc_sh.at[dst_v.at[j]], sem).wait()

        def body(jj, carry):
            a = 2 * jj
            b = a + 1
            gather_wait(a, 0, 0, gsem0)
            gather_wait(a, 1, 1, gsem1)
            scatter(a, 0, ssem0)
            gather_wait(b, 0, 2, gsem2)
            gather_wait(b, 1, 3, gsem3)
            scatter(b, 1, ssem1)

            @pl.when(jj < kp // 2 - 1)
            def _():
                scatter_wait(a, 0, ssem0)
                gather(a + 2, 0, 0, gsem0)
                gather(a + 2, 1, 1, gsem1)
                scatter_wait(b, 1, ssem1)
                gather(b + 2, 0, 2, gsem2)
                gather(b + 2, 1, 3, gsem3)

            return carry

        for phase in range(2):
            if phase:
                stage(phase, 0, srcp_hbm, src_v, pltpu.sync_copy)
                stage(phase, 1, dstp_hbm, dst_v, pltpu.sync_copy)
            gather(0, 0, 0, gsem0)
            gather(0, 1, 1, gsem1)
            gather(1, 0, 2, gsem2)
            gather(1, 1, 3, gsem3)
            lax.fori_loop(0, kp // 2, body, 0)
            scatter_wait(kp - 2, 0, ssem0)
            scatter_wait(kp - 1, 1, ssem1)
        plsc.subcore_barrier()

        @pl.when(cid == 0)
        def _():
            pltpu.sync_copy(acc_sh.at[pl.ds(sid * rpt, rpt)],
                            out0_hbm.at[pl.ds(sid * rpt, rpt)])

        @pl.when(cid == 1)
        def _():
            pltpu.sync_copy(acc_sh.at[pl.ds(sid * rpt, rpt)],
                            out1_hbm.at[pl.ds(sid * rpt, rpt)])

    return agg(x, edges_m, src_pad, dst_pad)


def _tc_mlp(x, p0, p1, W1, b1, g1, be1, W2, b2, g2, be2, W3, b3,
            gbn, bbn, Wm, bm, Wv, bv, block_rows):
    n, d = x.shape
    h_dim = W1.shape[1]
    grid = (pl.cdiv(n, block_rows),)
    isq = float(1.0 / np.sqrt(1.0 + BN_EPS))

    def mm(h, w):
        return lax.dot_general(h, w, (((1,), (0,)), ((), ())),
                               preferred_element_type=jnp.float32)

    def body(x_r, p0_r, p1_r, W1_r, b1_r, g1_r, be1_r,
             W2_r, b2_r, g2_r, be2_r, W3_r, b3_r, gbn_r, bbn_r,
             Wm_r, bm_r, Wv_r, bv_r, mean_r, var_r):
        h = x_r[...] + p0_r[...] + p1_r[...]
        s1 = g1_r[...] * isq
        h = jnp.maximum(mm(h, W1_r[...]) * s1 + (b1_r[...] * s1 + be1_r[...]),
                        0.0)
        s2 = g2_r[...] * isq
        h = jnp.maximum(mm(h, W2_r[...]) * s2 + (b2_r[...] * s2 + be2_r[...]),
                        0.0)
        h = jnp.maximum(mm(h, W3_r[...]) + b3_r[...], 0.0)
        h = h * (gbn_r[...] * isq) + bbn_r[...]
        mean_r[...] = mm(h, Wm_r[...]) + bm_r[...]
        var_r[...] = mm(h, Wv_r[...]) + bv_r[...]

    row_spec = pl.BlockSpec((block_rows, d), lambda i: (i, 0))
    w_spec = pl.BlockSpec((d, h_dim), lambda i: (0, 0))
    b_spec = pl.BlockSpec((h_dim,), lambda i: (0,))
    return pl.pallas_call(
        body,
        grid=grid,
        in_specs=[row_spec, row_spec, row_spec,
                  w_spec, b_spec, b_spec, b_spec,
                  w_spec, b_spec, b_spec, b_spec,
                  w_spec, b_spec,
                  b_spec, b_spec,
                  w_spec, b_spec, w_spec, b_spec],
        out_specs=(pl.BlockSpec((block_rows, h_dim), lambda i: (i, 0)),
                   pl.BlockSpec((block_rows, h_dim), lambda i: (i, 0))),
        out_shape=(jax.ShapeDtypeStruct((n, h_dim), jnp.float32),
                   jax.ShapeDtypeStruct((n, h_dim), jnp.float32)),
    )(x, p0, p1, W1, b1, g1, be1, W2, b2, g2, be2, W3, b3,
      gbn, bbn, Wm, bm, Wv, bv)


def kernel(x, edge_index, W1, b1, g1, be1, W2, b2, g2, be2, W3, b3,
           gbn, bbn, Wm, bm, Wv, bv):
    n, d = x.shape
    e = edge_index.shape[1]
    assert e % CH == 0

    rc = e // CH
    k = pl.cdiv(rc, NW)
    k += (-k) % 4
    n_acc = n + (-n) % (NS * CH)
    n_dummy = n_acc - n
    edges_m = edge_index.reshape(2, rc, CH)
    rib = (rc - (NW - 1) * k) // 8 * 8
    split = (NW - 1) * k + rib
    pc = NW * k - split
    ci = np.arange(pc - (rc - split), dtype=np.int32)[:, None]
    lane = np.arange(CH, dtype=np.int32)[None, :]
    src_pad = jnp.concatenate(
        [edges_m[0, split:], jnp.asarray((ci * CH + lane) % n)])
    dst_pad = jnp.concatenate(
        [edges_m[1, split:], jnp.asarray(n + (ci * 7 + lane) % n_dummy)])

    p0, p1 = _sc_edge_aggregate(x, edges_m, src_pad, dst_pad, n_acc, k)
    return _tc_mlp(x, p0, p1, W1, b1, g1, be1, W2, b2, g2, be2, W3, b3,
                   gbn, bbn, Wm, bm, Wv, bv, block_rows=2048)


# speedup vs baseline: 1.1119x; 1.1119x over previous
import functools

import jax
import jax.numpy as jnp
import numpy as np
from jax import lax
from jax.experimental import pallas as pl
from jax.experimental.pallas import tpu as pltpu
from jax.experimental.pallas import tpu_sc as plsc

NC = 2
NS = 16
NW = NC * NS
CH = 64
NBUF = 4
BN_EPS = 1e-3


def _sc_edge_aggregate(x, edges_m, src_pad, dst_pad, n_acc, k):
    n, D = x.shape
    rc = edges_m.shape[1]
    rpt = n_acc // NS
    kp = 40
    ph = k // kp
    bw = NW - 1
    assert bw * k < rc <= NW * k
    rib = (rc - bw * k) // 8 * 8
    mesh = plsc.VectorSubcoreMesh(core_axis_name="c", subcore_axis_name="s")

    @functools.partial(
        pl.kernel,
        out_type=(
            jax.ShapeDtypeStruct((n_acc, D), jnp.float32),
            jax.ShapeDtypeStruct((n_acc, D), jnp.float32),
        ),
        mesh=mesh,
        scratch_types=[
            pltpu.VMEM((kp, CH), jnp.int32),
            pltpu.VMEM((kp, CH), jnp.int32),
            *[pltpu.VMEM((CH, D), jnp.float32) for _ in range(NBUF)],
            pltpu.SemaphoreType.DMA,
            *[pltpu.SemaphoreType.DMA for _ in range(NBUF)],
            *[pltpu.SemaphoreType.DMA for _ in range(NBUF)],
            pltpu.VMEM_SHARED((n_acc, D), jnp.float32),
        ],
    )
    def agg(x_hbm, edges_hbm, srcp_hbm, dstp_hbm,
            out0_hbm, out1_hbm,
            src_v, dst_v, *bufsem):
        rows = bufsem[:NBUF]
        isem = bufsem[NBUF]
        gsem = bufsem[NBUF + 1:2 * NBUF + 1]
        ssem = bufsem[2 * NBUF + 1:3 * NBUF + 1]
        acc_sh = bufsem[3 * NBUF + 1]
        cid = lax.axis_index("c")
        sid = lax.axis_index("s")
        wid = sid * NC + cid

        def stage(phase, row, pad_hbm, buf, op):
            r_lo = min(rib, phase * kp)
            r_ph = min(rib - r_lo, kp)
            p_lo = phase * kp - r_lo

            @pl.when(wid < bw)
            def _():
                op(edges_hbm.at[row, pl.ds(wid * k + phase * kp, kp)], buf)

            @pl.when(wid == bw)
            def _():
                if r_ph:
                    op(edges_hbm.at[row, pl.ds(bw * k + r_lo, r_ph)],
                       buf.at[pl.ds(0, r_ph)])
                if kp - r_ph:
                    op(pad_hbm.at[pl.ds(p_lo, kp - r_ph)],
                       buf.at[pl.ds(r_ph, kp - r_ph)])

        def istart(s, d):
            pltpu.async_copy(s, d, isem)

        def iwait(s, d):
            pltpu.make_async_copy(s, d, isem).wait()

        stage(0, 0, srcp_hbm, src_v, istart)
        stage(0, 1, dstp_hbm, dst_v, istart)

        zvec = jnp.zeros((16,), jnp.float32)

        def zrow(i, carry):
            for l in range(D // 16):
                rows[0][i, pl.ds(l * 16, 16)] = zvec
            return carry

        lax.fori_loop(0, CH, zrow, 0)
        for r in range(rpt // CH):
            pltpu.async_copy(rows[0],
                             acc_sh.at[pl.ds(sid * rpt + r * CH, CH)],
                             ssem[r % NBUF])
        for r in range(rpt // CH):
            pltpu.make_async_copy(
                rows[0], acc_sh.at[pl.ds(sid * rpt + r * CH, CH)],
                ssem[r % NBUF]).wait()

        stage(0, 0, srcp_hbm, src_v, iwait)
        stage(0, 1, dstp_hbm, dst_v, iwait)
        plsc.subcore_barrier()

        def gather(j, buf, sem):
            pltpu.async_copy(x_hbm.at[src_v.at[j]], buf, sem)

        def gather_wait(j, buf, sem):
            pltpu.make_async_copy(x_hbm.at[src_v.at[j]], buf, sem).wait()

        def scatter(j, buf, sem):
            pltpu.async_copy(buf, acc_sh.at[dst_v.at[j]], sem, add=True)

        def scatter_wait(j, buf, sem):
            pltpu.make_async_copy(buf, acc_sh.at[dst_v.at[j]], sem).wait()

        def body(jj, carry):
            base = NBUF * jj
            for i in range(NBUF):
                gather_wait(base + i, rows[i], gsem[i])
                scatter(base + i, rows[i], ssem[i])

            @pl.when(jj < kp // NBUF - 1)
            def _():
                for i in range(NBUF):
                    scatter_wait(base + i, rows[i], ssem[i])
                    gather(base + NBUF + i, rows[i], gsem[i])

            return carry

        for phase in range(ph):
            if phase:
                stage(phase, 0, srcp_hbm, src_v, pltpu.sync_copy)
                stage(phase, 1, dstp_hbm, dst_v, pltpu.sync_copy)
            for i in range(NBUF):
                gather(i, rows[i], gsem[i])
            lax.fori_loop(0, kp // NBUF, body, 0)
            for i in range(NBUF):
                scatter_wait(kp - NBUF + i, rows[i], ssem[i])
        plsc.subcore_barrier()

        @pl.when(cid == 0)
        def _():
            pltpu.sync_copy(acc_sh.at[pl.ds(sid * rpt, rpt)],
                            out0_hbm.at[pl.ds(sid * rpt, rpt)])

        @pl.when(cid == 1)
        def _():
            pltpu.sync_copy(acc_sh.at[pl.ds(sid * rpt, rpt)],
                            out1_hbm.at[pl.ds(sid * rpt, rpt)])

    return agg(x, edges_m, src_pad, dst_pad)


def _tc_mlp(x, p0, p1, W1, b1, g1, be1, W2, b2, g2, be2, W3, b3,
            gbn, bbn, Wm, bm, Wv, bv, block_rows):
    n, d = x.shape
    h_dim = W1.shape[1]
    grid = (pl.cdiv(n, block_rows),)
    isq = float(1.0 / np.sqrt(1.0 + BN_EPS))

    def mm(h, w):
        return lax.dot_general(h, w, (((1,), (0,)), ((), ())),
                               preferred_element_type=jnp.float32)

    def body(x_r, p0_r, p1_r, W1_r, b1_r, g1_r, be1_r,
             W2_r, b2_r, g2_r, be2_r, W3_r, b3_r, gbn_r, bbn_r,
             Wm_r, bm_r, Wv_r, bv_r, mean_r, var_r):
        h = x_r[...] + p0_r[...] + p1_r[...]
        s1 = g1_r[...] * isq
        h = jnp.maximum(mm(h, W1_r[...]) * s1 + (b1_r[...] * s1 + be1_r[...]),
                        0.0)
        s2 = g2_r[...] * isq
        h = jnp.maximum(mm(h, W2_r[...]) * s2 + (b2_r[...] * s2 + be2_r[...]),
                        0.0)
        h = jnp.maximum(mm(h, W3_r[...]) + b3_r[...], 0.0)
        h = h * (gbn_r[...] * isq) + bbn_r[...]
        mean_r[...] = mm(h, Wm_r[...]) + bm_r[...]
        var_r[...] = mm(h, Wv_r[...]) + bv_r[...]

    row_spec = pl.BlockSpec((block_rows, d), lambda i: (i, 0))
    w_spec = pl.BlockSpec((d, h_dim), lambda i: (0, 0))
    b_spec = pl.BlockSpec((h_dim,), lambda i: (0,))
    return pl.pallas_call(
        body,
        grid=grid,
        in_specs=[row_spec, row_spec, row_spec,
                  w_spec, b_spec, b_spec, b_spec,
                  w_spec, b_spec, b_spec, b_spec,
                  w_spec, b_spec,
                  b_spec, b_spec,
                  w_spec, b_spec, w_spec, b_spec],
        out_specs=(pl.BlockSpec((block_rows, h_dim), lambda i: (i, 0)),
                   pl.BlockSpec((block_rows, h_dim), lambda i: (i, 0))),
        out_shape=(jax.ShapeDtypeStruct((n, h_dim), jnp.float32),
                   jax.ShapeDtypeStruct((n, h_dim), jnp.float32)),
    )(x, p0, p1, W1, b1, g1, be1, W2, b2, g2, be2, W3, b3,
      gbn, bbn, Wm, bm, Wv, bv)


def kernel(x, edge_index, W1, b1, g1, be1, W2, b2, g2, be2, W3, b3,
           gbn, bbn, Wm, bm, Wv, bv):
    n, d = x.shape
    e = edge_index.shape[1]
    assert e % CH == 0

    rc = e // CH
    k = pl.cdiv(rc, NW)
    k += (-k) % 40
    n_acc = n + (-n) % (NS * CH)
    n_dummy = n_acc - n
    edges_m = edge_index.reshape(2, rc, CH)
    rib = (rc - (NW - 1) * k) // 8 * 8
    split = (NW - 1) * k + rib
    pc = NW * k - split
    ci = np.arange(pc - (rc - split), dtype=np.int32)[:, None]
    lane = np.arange(CH, dtype=np.int32)[None, :]
    src_pad = jnp.concatenate(
        [edges_m[0, split:], jnp.asarray((ci * CH + lane) % n)])
    dst_pad = jnp.concatenate(
        [edges_m[1, split:], jnp.asarray(n + (ci * 7 + lane) % n_dummy)])

    p0, p1 = _sc_edge_aggregate(x, edges_m, src_pad, dst_pad, n_acc, k)
    return _tc_mlp(x, p0, p1, W1, b1, g1, be1, W2, b2, g2, be2, W3, b3,
                   gbn, bbn, Wm, bm, Wv, bv, block_rows=1024)


# speedup vs baseline: 1.1427x; 1.0277x over previous
import functools

import jax
import jax.numpy as jnp
import numpy as np
from jax import lax
from jax.experimental import pallas as pl
from jax.experimental.pallas import tpu as pltpu
from jax.experimental.pallas import tpu_sc as plsc

NC = 2
NS = 16
NW = NC * NS
CH = 64
NBUF = 4
BN_EPS = 1e-3


def _sc_edge_aggregate(x, edges_m, src_pad, dst_pad, n_acc, k):
    n, D = x.shape
    rc = edges_m.shape[1]
    rpt = n_acc // NS
    kp = 40
    ph = k // kp
    bw = NW - 1
    assert bw * k < rc <= NW * k
    rib = (rc - bw * k) // 8 * 8
    mesh = plsc.VectorSubcoreMesh(core_axis_name="c", subcore_axis_name="s")

    @functools.partial(
        pl.kernel,
        out_type=(
            jax.ShapeDtypeStruct((n_acc, D), jnp.float32),
            jax.ShapeDtypeStruct((n_acc, D), jnp.float32),
        ),
        mesh=mesh,
        scratch_types=[
            pltpu.VMEM((kp, CH), jnp.int32),
            pltpu.VMEM((kp, CH), jnp.int32),
            *[pltpu.VMEM((CH, D), jnp.float32) for _ in range(NBUF)],
            pltpu.SemaphoreType.DMA,
            *[pltpu.SemaphoreType.DMA for _ in range(NBUF)],
            *[pltpu.SemaphoreType.DMA for _ in range(NBUF)],
            pltpu.VMEM_SHARED((n_acc, D), jnp.float32),
        ],
    )
    def agg(x_hbm, edges_hbm, srcp_hbm, dstp_hbm,
            out0_hbm, out1_hbm,
            src_v, dst_v, *bufsem):
        rows = bufsem[:NBUF]
        isem = bufsem[NBUF]
        gsem = bufsem[NBUF + 1:2 * NBUF + 1]
        ssem = bufsem[2 * NBUF + 1:3 * NBUF + 1]
        acc_sh = bufsem[3 * NBUF + 1]
        cid = lax.axis_index("c")
        sid = lax.axis_index("s")
        wid = sid * NC + cid

        def stage(phase, row, pad_hbm, buf, op):
            r_lo = min(rib, phase * kp)
            r_ph = min(rib - r_lo, kp)
            p_lo = phase * kp - r_lo

            @pl.when(wid < bw)
            def _():
                op(edges_hbm.at[row, pl.ds(wid * k + phase * kp, kp)], buf)

            @pl.when(wid == bw)
            def _():
                if r_ph:
                    op(edges_hbm.at[row, pl.ds(bw * k + r_lo, r_ph)],
                       buf.at[pl.ds(0, r_ph)])
                if kp - r_ph:
                    op(pad_hbm.at[pl.ds(p_lo, kp - r_ph)],
                       buf.at[pl.ds(r_ph, kp - r_ph)])

        def istart(s, d):
            pltpu.async_copy(s, d, isem)

        def iwait(s, d):
            pltpu.make_async_copy(s, d, isem).wait()

        stage(0, 0, srcp_hbm, src_v, istart)
        stage(0, 1, dstp_hbm, dst_v, istart)

        zvec = jnp.zeros((16,), jnp.float32)

        def zrow(i, carry):
            for l in range(D // 16):
                rows[0][i, pl.ds(l * 16, 16)] = zvec
            return carry

        lax.fori_loop(0, CH, zrow, 0)
        for r in range(rpt // CH):
            pltpu.async_copy(rows[0],
                             acc_sh.at[pl.ds(sid * rpt + r * CH, CH)],
                             ssem[r % NBUF])
        for r in range(rpt // CH):
            pltpu.make_async_copy(
                rows[0], acc_sh.at[pl.ds(sid * rpt + r * CH, CH)],
                ssem[r % NBUF]).wait()

        stage(0, 0, srcp_hbm, src_v, iwait)
        stage(0, 1, dstp_hbm, dst_v, iwait)
        plsc.subcore_barrier()

        def gather(j, buf, sem):
            pltpu.async_copy(x_hbm.at[src_v.at[j]], buf, sem)

        def gather_wait(j, buf, sem):
            pltpu.make_async_copy(x_hbm.at[src_v.at[j]], buf, sem).wait()

        def scatter(j, buf, sem):
            pltpu.async_copy(buf, acc_sh.at[dst_v.at[j]], sem, add=True)

        def scatter_wait(j, buf, sem):
            pltpu.make_async_copy(buf, acc_sh.at[dst_v.at[j]], sem).wait()

        def body(jj, carry):
            base = NBUF * jj
            for i in range(NBUF):
                gather_wait(base + i, rows[i], gsem[i])
                scatter(base + i, rows[i], ssem[i])

            @pl.when(jj < kp // NBUF - 1)
            def _():
                for i in range(NBUF):
                    scatter_wait(base + i, rows[i], ssem[i])
                    gather(base + NBUF + i, rows[i], gsem[i])

            return carry

        for phase in range(ph):
            if phase:
                stage(phase, 0, srcp_hbm, src_v, pltpu.sync_copy)
                stage(phase, 1, dstp_hbm, dst_v, pltpu.sync_copy)
            for i in range(NBUF):
                gather(i, rows[i], gsem[i])
            lax.fori_loop(0, kp // NBUF, body, 0)
            for i in range(NBUF):
                scatter_wait(kp - NBUF + i, rows[i], ssem[i])
        plsc.subcore_barrier()

        @pl.when(cid == 0)
        def _():
            pltpu.sync_copy(acc_sh.at[pl.ds(sid * rpt, rpt)],
                            out0_hbm.at[pl.ds(sid * rpt, rpt)])

        @pl.when(cid == 1)
        def _():
            pltpu.sync_copy(acc_sh.at[pl.ds(sid * rpt, rpt)],
                            out1_hbm.at[pl.ds(sid * rpt, rpt)])

    return agg(x, edges_m, src_pad, dst_pad)


def _tc_mlp(x, p0, p1, W1, b1, g1, be1, W2, b2, g2, be2, W3, b3,
            gbn, bbn, Wm, bm, Wv, bv, block_rows):
    n, d = x.shape
    h_dim = W1.shape[1]
    grid = (pl.cdiv(n, block_rows),)
    isq = float(1.0 / np.sqrt(1.0 + BN_EPS))

    def mm(h, w):
        return lax.dot_general(h, w, (((1,), (0,)), ((), ())),
                               preferred_element_type=jnp.float32)

    def body(x_r, p0_r, p1_r, W1_r, b1_r, g1_r, be1_r,
             W2_r, b2_r, g2_r, be2_r, W3_r, b3_r, gbn_r, bbn_r,
             Wm_r, bm_r, Wv_r, bv_r, mean_r, var_r):
        h = x_r[...] + p0_r[...] + p1_r[...]
        s1 = g1_r[...] * isq
        h = jnp.maximum(mm(h, W1_r[...]) * s1 + (b1_r[...] * s1 + be1_r[...]),
                        0.0)
        s2 = g2_r[...] * isq
        h = jnp.maximum(mm(h, W2_r[...]) * s2 + (b2_r[...] * s2 + be2_r[...]),
                        0.0)
        h = jnp.maximum(mm(h, W3_r[...]) + b3_r[...], 0.0)
        h = h * (gbn_r[...] * isq) + bbn_r[...]
        mean_r[...] = mm(h, Wm_r[...]) + bm_r[...]
        var_r[...] = mm(h, Wv_r[...]) + bv_r[...]

    row_spec = pl.BlockSpec((block_rows, d), lambda i: (i, 0))
    w_spec = pl.BlockSpec((d, h_dim), lambda i: (0, 0))
    b_spec = pl.BlockSpec((h_dim,), lambda i: (0,))
    return pl.pallas_call(
        body,
        grid=grid,
        in_specs=[row_spec, row_spec, row_spec,
                  w_spec, b_spec, b_spec, b_spec,
                  w_spec, b_spec, b_spec, b_spec,
                  w_spec, b_spec,
                  b_spec, b_spec,
                  w_spec, b_spec, w_spec, b_spec],
        out_specs=(pl.BlockSpec((block_rows, h_dim), lambda i: (i, 0)),
                   pl.BlockSpec((block_rows, h_dim), lambda i: (i, 0))),
        out_shape=(jax.ShapeDtypeStruct((n, h_dim), jnp.float32),
                   jax.ShapeDtypeStruct((n, h_dim), jnp.float32)),
    )(x, p0, p1, W1, b1, g1, be1, W2, b2, g2, be2, W3, b3,
      gbn, bbn, Wm, bm, Wv, bv)


def kernel(x, edge_index, W1, b1, g1, be1, W2, b2, g2, be2, W3, b3,
           gbn, bbn, Wm, bm, Wv, bv):
    n, d = x.shape
    e = edge_index.shape[1]
    assert e % CH == 0

    rc = e // CH
    k = pl.cdiv(rc, NW)
    k += (-k) % 40
    n_acc = n + (-n) % (NS * CH)
    n_dummy = n_acc - n
    edges_m = edge_index.reshape(2, rc, CH)
    rib = (rc - (NW - 1) * k) // 8 * 8
    split = (NW - 1) * k + rib
    pc = NW * k - split
    ci = np.arange(pc - (rc - split), dtype=np.int32)[:, None]
    lane = np.arange(CH, dtype=np.int32)[None, :]
    src_pad = jnp.concatenate(
        [edges_m[0, split:], jnp.asarray((ci * CH + lane) % n)])
    dst_pad = jnp.concatenate(
        [edges_m[1, split:], jnp.asarray(n + (ci * 7 + lane) % n_dummy)])

    p0, p1 = _sc_edge_aggregate(x, edges_m, src_pad, dst_pad, n_acc, k)
    return _tc_mlp(x, p0, p1, W1, b1, g1, be1, W2, b2, g2, be2, W3, b3,
                   gbn, bbn, Wm, bm, Wv, bv, block_rows=2048)
